# baseline (speedup 1.0000x reference)
import functools

import jax
import jax.numpy as jnp
from jax import lax
from jax.experimental import pallas as pl
from jax.experimental.pallas import tpu as pltpu
from jax.experimental.pallas import tpu_sc as plsc

NN = 10000
NE = 2048
NNZ = 160000
D = 256
H = 128
NP = 10240
TILES = 16
PER_TILE = NNZ // TILES
CHUNK = 80
SEC = 2000
NSEC = PER_TILE // SEC
OC = 32


def _make_seg_kernel(n_rows, s_pad, has_gval, apply_relu, chunk, dbl):
  rows_t = s_pad // TILES
  n_oc = rows_t // OC
  sec_chunks = SEC // chunk

  mesh = plsc.VectorSubcoreMesh(core_axis_name="c", subcore_axis_name="s")

  @functools.partial(
      pl.kernel,
      out_type=(jax.ShapeDtypeStruct((2 * s_pad, H), jnp.float32),
                jax.ShapeDtypeStruct((2, TILES, s_pad), jnp.float32)),
      mesh=mesh,
      compiler_params=pltpu.CompilerParams(needs_layout_passes=False),
      scratch_types=[
          pltpu.VMEM((SEC,), jnp.int32),
          pltpu.VMEM((SEC,), jnp.int32),
          pltpu.VMEM(((2 if dbl else 1) * chunk, H), jnp.float32),
          pltpu.VMEM((2 * OC, H), jnp.float32),
          pltpu.VMEM((s_pad,), jnp.float32),
          pltpu.VMEM((rows_t,), jnp.float32),
          pltpu.VMEM((rows_t,), jnp.float32),
      ] + ([pltpu.VMEM((NE,), jnp.float32)] if has_gval else []) + [
          pltpu.VMEM_SHARED((s_pad, H), jnp.float32),
          pltpu.SemaphoreType.DMA,
          pltpu.SemaphoreType.DMA,
      ],
  )
  def seg(table, idxg4, idxs4, *args):
    (gval, out, scal_pub,
     idxg_v, idxs_v, rows_v, zbuf, scal_l, den_inv, tmp_v) = args[:10]
    rest = args[10:]
    if has_gval:
      gval_v, acc, sem_a, sem_b = rest
    else:
      acc, sem_a, sem_b = rest
      gval_v = None
    sem_c, sem_d = sem_a, sem_b
    c = lax.axis_index("c")
    s = lax.axis_index("s")
    z16 = jnp.zeros((16,), jnp.float32)

    def zero_zbuf(i, carry):
      for j in range(H // 16):
        zbuf[i, pl.ds(j * 16, 16)] = z16
      return carry
    lax.fori_loop(0, OC, zero_zbuf, 0)

    def zero_scal(i, carry):
      scal_l[pl.ds(i * 16, 16)] = z16
      return carry
    lax.fori_loop(0, s_pad // 16, zero_scal, 0)

    def zero_acc(q, carry):
      pltpu.sync_copy(zbuf.at[pl.ds(0, OC)],
                      acc.at[pl.ds(s * rows_t + q * OC, OC)])
      return carry
    lax.fori_loop(0, n_oc, zero_acc, 0)
    plsc.subcore_barrier()

    if has_gval:
      pltpu.sync_copy(gval, gval_v)
    goff = c * n_rows

    def g_start(k, buf, sem):
      pltpu.async_copy(table.at[idxg_v.at[pl.ds(k * chunk, chunk)]],
                       buf, sem)

    def g_wait(buf, sem):
      pltpu.make_async_copy(table.at[idxg_v.at[pl.ds(0, chunk)]],
                            buf, sem).wait()

    def scat(k, buf):
      pltpu.sync_copy(buf, acc.at[idxs_v.at[pl.ds(k * chunk, chunk)]],
                      add=True)

    buf_a = rows_v.at[pl.ds(0, chunk)]
    buf_b = rows_v.at[pl.ds(chunk, chunk)] if dbl else None

    def section(sec):
      pltpu.sync_copy(idxg4.at[sec, c, s], idxg_v)
      pltpu.sync_copy(idxs4.at[sec, s], idxs_v)
      if dbl:
        g_start(0, buf_a, sem_a)

      def row_chunk(k, carry2):
        for j in range(chunk // 16):
          si = idxs_v[pl.ds(k * chunk + j * 16, 16)]
          if has_gval:
            gi = idxg_v[pl.ds(k * chunk + j * 16, 16)] - goff
            vals = plsc.load_gather(gval_v, [gi])
          else:
            vals = jnp.ones((16,), jnp.float32)
          plsc.addupdate_scatter(scal_l, [si], vals)

        if dbl:
          def step(buf, sem, obuf, osem):
            g_wait(buf, sem)
            @pl.when(k + 1 < sec_chunks)
            def _():
              g_start(k + 1, obuf, osem)
            scat(k, buf)
          @pl.when(lax.rem(k, 2) == 0)
          def _():
            step(buf_a, sem_a, buf_b, sem_b)
          @pl.when(lax.rem(k, 2) == 1)
          def _():
            step(buf_b, sem_b, buf_a, sem_a)
        else:
          g_start(k, rows_v, sem_a)
          g_wait(rows_v, sem_a)
          scat(k, rows_v)
        return carry2
      lax.fori_loop(0, sec_chunks, row_chunk, 0)

    for sec in range(NSEC):
      section(sec)

    pltpu.sync_copy(scal_l, scal_pub.at[c, s])

    plsc.subcore_barrier()

    pltpu.sync_copy(scal_pub.at[c, 0, pl.ds(s * rows_t, rows_t)], den_inv)
    for t2 in range(1, TILES):
      pltpu.sync_copy(scal_pub.at[c, t2, pl.ds(s * rows_t, rows_t)], tmp_v)
      def red(i, carry):
        den_inv[pl.ds(i * 16, 16)] = (den_inv[pl.ds(i * 16, 16)]
                                      + tmp_v[pl.ds(i * 16, 16)])
        return carry
      lax.fori_loop(0, rows_t // 16, red, 0)
    def inv_loop(i, carry):
      dv = den_inv[pl.ds(i * 16, 16)]
      den_inv[pl.ds(i * 16, 16)] = 1.0 / jnp.where(dv > 0.0, dv, 1.0)
      return carry
    lax.fori_loop(0, rows_t // 16, inv_loop, 0)

    zb = [zbuf.at[pl.ds(0, OC)], zbuf.at[pl.ds(OC, OC)]]

    def o_in_start(q, p, sem):
      pltpu.async_copy(acc.at[pl.ds(s * rows_t + q * OC, OC)], zb[p], sem)

    def o_in_wait(p, sem):
      pltpu.make_async_copy(acc.at[pl.ds(0, OC)], zb[p], sem).wait()

    def o_step(q, p, sem, osem):
      o_in_wait(p, sem)
      @pl.when(q + 1 < n_oc)
      def _():
        o_in_start(q + 1, 1 - p, osem)
      def out_row(i, c2):
        inv = plsc.load_gather(
            den_inv, [jnp.zeros((16,), jnp.int32) + (q * OC + i)])
        for j in range(H // 16):
          v = zbuf[p * OC + i, pl.ds(j * 16, 16)] * inv
          if apply_relu:
            v = jnp.maximum(v, 0.0)
          zbuf[p * OC + i, pl.ds(j * 16, 16)] = v
        return c2
      lax.fori_loop(0, OC, out_row, 0)
      pltpu.sync_copy(zb[p], out.at[pl.ds(c * s_pad + s * rows_t + q * OC,
                                          OC)])

    o_in_start(0, 0, sem_c)
    def out_chunk(q, carry):
      @pl.when(lax.rem(q, 2) == 0)
      def _():
        o_step(q, 0, sem_c, sem_d)
      @pl.when(lax.rem(q, 2) == 1)
      def _():
        o_step(q, 1, sem_d, sem_c)
      return carry
    lax.fori_loop(0, n_oc, out_chunk, 0)

  return seg


_seg_v2e = _make_seg_kernel(NP, NE, has_gval=False, apply_relu=False,
                            chunk=80, dbl=True)
_seg_e2v = _make_seg_kernel(NE, NP, has_gval=True, apply_relu=True,
                            chunk=80, dbl=True)


def _mm_body(x0, x1, w, b, o):
  acc = jnp.dot(x0[...], w[:H, :], preferred_element_type=jnp.float32)
  acc += jnp.dot(x1[...], w[H:, :], preferred_element_type=jnp.float32)
  o[...] = acc + b[...]


def _matmul_halves(xh, w, b):
  mb = 512
  grid = (NP // mb, 2)
  return pl.pallas_call(
      _mm_body,
      grid=grid,
      in_specs=[
          pl.BlockSpec((mb, H), lambda m, h: (m, 0)),
          pl.BlockSpec((mb, H), lambda m, h: (NP // mb + m, 0)),
          pl.BlockSpec((D, H), lambda m, h: (0, h)),
          pl.BlockSpec((1, H), lambda m, h: (0, h)),
      ],
      out_specs=pl.BlockSpec((mb, H), lambda m, h: (h * (NP // mb) + m, 0)),
      out_shape=jax.ShapeDtypeStruct((2 * NP, H), jnp.float32),
  )(xh, xh, w, b.reshape(1, D))


def _edge_body(y, a0, a1, z, e):
  y0 = y[:NE, :]
  y1 = y[NE:, :]
  alpha = (jnp.sum(y0 * a0[...], axis=1, keepdims=True)
           + jnp.sum(y1 * a1[...], axis=1, keepdims=True))
  sc = jnp.where(alpha > 0.0, alpha, 0.2 * alpha)
  ev = jnp.exp(sc - jnp.max(sc))
  z[:NE, :] = y0 * ev
  z[NE:, :] = y1 * ev
  e[...] = ev


def _edge_call(y, a):
  zh, e = pl.pallas_call(
      _edge_body,
      in_specs=[
          pl.BlockSpec((2 * NE, H), lambda: (0, 0)),
          pl.BlockSpec((1, H), lambda: (0, 0)),
          pl.BlockSpec((1, H), lambda: (0, 0)),
      ],
      out_specs=[
          pl.BlockSpec((2 * NE, H), lambda: (0, 0)),
          pl.BlockSpec((NE, 1), lambda: (0, 0)),
      ],
      out_shape=[
          jax.ShapeDtypeStruct((2 * NE, H), jnp.float32),
          jax.ShapeDtypeStruct((NE, 1), jnp.float32),
      ],
  )(y, a[:H].reshape(1, H), a[H:].reshape(1, H))
  return zh, e.reshape(NE)


def _layer(xh, nig4, ni4, eig4, ei4, gdummy, w, b, a):
  xph = _matmul_halves(xh, w, b)
  yh, _ = _seg_v2e(xph, nig4, ei4, gdummy)
  zh, e = _edge_call(yh, a)
  xo, _ = _seg_e2v(zh, eig4, ni4, e)
  return xo


def kernel(x, node_idx, edge_idx, W1, b1, a1, W2, b2, a2):
  ni = node_idx.astype(jnp.int32).reshape(TILES, NSEC, SEC)
  ei = edge_idx.astype(jnp.int32).reshape(TILES, NSEC, SEC)
  ni4 = ni.transpose(1, 0, 2)
  ei4 = ei.transpose(1, 0, 2)
  nig4 = jnp.stack([ni4, ni4 + NP], axis=1)
  eig4 = jnp.stack([ei4, ei4 + NE], axis=1)
  pad = jnp.zeros((NP - NN, H), jnp.float32)
  xh = jnp.concatenate([x[:, :H], pad, x[:, H:], pad], axis=0)
  gdummy = jnp.zeros((NE,), jnp.float32)

  h = _layer(xh, nig4, ni4, eig4, ei4, gdummy, W1, b1, a1)
  h = _layer(h, nig4, ni4, eig4, ei4, gdummy, W2, b2, a2)
  return jnp.concatenate([h[:NN, :], h[NP:NP + NN, :]], axis=1)


# speedup vs baseline: 1.0094x; 1.0094x over previous
import functools

import jax
import jax.numpy as jnp
from jax import lax
from jax.experimental import pallas as pl
from jax.experimental.pallas import tpu as pltpu
from jax.experimental.pallas import tpu_sc as plsc

NN = 10000
NE = 2048
NNZ = 160000
D = 256
H = 128
NP = 10240
TILES = 16
PER_TILE = NNZ // TILES
CHUNK = 80
SEC = 2000
NSEC = PER_TILE // SEC
OC = 32


def _make_seg_kernel(n_rows, s_pad, has_gval, apply_relu, chunk, dbl):
  rows_t = s_pad // TILES
  n_oc = rows_t // OC
  sec_chunks = SEC // chunk

  mesh = plsc.VectorSubcoreMesh(core_axis_name="c", subcore_axis_name="s")

  @functools.partial(
      pl.kernel,
      out_type=(jax.ShapeDtypeStruct((2 * s_pad, H), jnp.float32),
                jax.ShapeDtypeStruct((2, TILES, s_pad), jnp.float32)),
      mesh=mesh,
      compiler_params=pltpu.CompilerParams(needs_layout_passes=False),
      scratch_types=[
          pltpu.VMEM((SEC,), jnp.int32),
          pltpu.VMEM((SEC,), jnp.int32),
          pltpu.VMEM(((2 if dbl else 1) * chunk, H), jnp.float32),
          pltpu.VMEM((2 * OC, H), jnp.float32),
          pltpu.VMEM((s_pad,), jnp.float32),
          pltpu.VMEM((rows_t,), jnp.float32),
          pltpu.VMEM((rows_t,), jnp.float32),
      ] + ([pltpu.VMEM((NE,), jnp.float32)] if has_gval else []) + [
          pltpu.VMEM_SHARED((s_pad, H), jnp.float32),
          pltpu.SemaphoreType.DMA,
          pltpu.SemaphoreType.DMA,
      ],
  )
  def seg(table, idxg4, idxs4, *args):
    (gval, out, scal_pub,
     idxg_v, idxs_v, rows_v, zbuf, scal_l, den_inv, tmp_v) = args[:10]
    rest = args[10:]
    if has_gval:
      gval_v, acc, sem_a, sem_b = rest
    else:
      acc, sem_a, sem_b = rest
      gval_v = None
    sem_c, sem_d = sem_a, sem_b
    c = lax.axis_index("c")
    s = lax.axis_index("s")
    z16 = jnp.zeros((16,), jnp.float32)

    def zero_zbuf(i, carry):
      for j in range(H // 16):
        zbuf[i, pl.ds(j * 16, 16)] = z16
      return carry
    lax.fori_loop(0, OC, zero_zbuf, 0)

    def zero_scal(i, carry):
      scal_l[pl.ds(i * 16, 16)] = z16
      return carry
    lax.fori_loop(0, s_pad // 16, zero_scal, 0)

    def zero_acc(q, carry):
      pltpu.sync_copy(zbuf.at[pl.ds(0, OC)],
                      acc.at[pl.ds(s * rows_t + q * OC, OC)])
      return carry
    lax.fori_loop(0, n_oc, zero_acc, 0)
    plsc.subcore_barrier()

    if has_gval:
      pltpu.sync_copy(gval, gval_v)
    goff = c * n_rows

    def g_start(k, buf, sem):
      pltpu.async_copy(table.at[idxg_v.at[pl.ds(k * chunk, chunk)]],
                       buf, sem)

    def g_wait(buf, sem):
      pltpu.make_async_copy(table.at[idxg_v.at[pl.ds(0, chunk)]],
                            buf, sem).wait()

    def scat(k, buf):
      pltpu.sync_copy(buf, acc.at[idxs_v.at[pl.ds(k * chunk, chunk)]],
                      add=True)

    buf_a = rows_v.at[pl.ds(0, chunk)]
    buf_b = rows_v.at[pl.ds(chunk, chunk)] if dbl else None

    def section(sec):
      pltpu.sync_copy(idxg4.at[c, sec, s], idxg_v)
      pltpu.sync_copy(idxs4.at[sec, s], idxs_v)
      if dbl:
        g_start(0, buf_a, sem_a)

      def row_chunk(k, carry2):
        for j in range(chunk // 16):
          si = idxs_v[pl.ds(k * chunk + j * 16, 16)]
          if has_gval:
            gi = idxg_v[pl.ds(k * chunk + j * 16, 16)] - goff
            vals = plsc.load_gather(gval_v, [gi])
          else:
            vals = jnp.ones((16,), jnp.float32)
          plsc.addupdate_scatter(scal_l, [si], vals)

        if dbl:
          def step(buf, sem, obuf, osem):
            g_wait(buf, sem)
            @pl.when(k + 1 < sec_chunks)
            def _():
              g_start(k + 1, obuf, osem)
            scat(k, buf)
          @pl.when(lax.rem(k, 2) == 0)
          def _():
            step(buf_a, sem_a, buf_b, sem_b)
          @pl.when(lax.rem(k, 2) == 1)
          def _():
            step(buf_b, sem_b, buf_a, sem_a)
        else:
          g_start(k, rows_v, sem_a)
          g_wait(rows_v, sem_a)
          scat(k, rows_v)
        return carry2
      lax.fori_loop(0, sec_chunks, row_chunk, 0)

    for sec in range(NSEC):
      section(sec)

    pltpu.sync_copy(scal_l, scal_pub.at[c, s])

    plsc.subcore_barrier()

    pltpu.sync_copy(scal_pub.at[c, 0, pl.ds(s * rows_t, rows_t)], den_inv)
    for t2 in range(1, TILES):
      pltpu.sync_copy(scal_pub.at[c, t2, pl.ds(s * rows_t, rows_t)], tmp_v)
      def red(i, carry):
        den_inv[pl.ds(i * 16, 16)] = (den_inv[pl.ds(i * 16, 16)]
                                      + tmp_v[pl.ds(i * 16, 16)])
        return carry
      lax.fori_loop(0, rows_t // 16, red, 0)
    def inv_loop(i, carry):
      dv = den_inv[pl.ds(i * 16, 16)]
      den_inv[pl.ds(i * 16, 16)] = 1.0 / jnp.where(dv > 0.0, dv, 1.0)
      return carry
    lax.fori_loop(0, rows_t // 16, inv_loop, 0)

    zb = [zbuf.at[pl.ds(0, OC)], zbuf.at[pl.ds(OC, OC)]]

    def o_in_start(q, p, sem):
      pltpu.async_copy(acc.at[pl.ds(s * rows_t + q * OC, OC)], zb[p], sem)

    def o_in_wait(p, sem):
      pltpu.make_async_copy(acc.at[pl.ds(0, OC)], zb[p], sem).wait()

    def o_step(q, p, sem, osem):
      o_in_wait(p, sem)
      @pl.when(q + 1 < n_oc)
      def _():
        o_in_start(q + 1, 1 - p, osem)
      def out_row(i, c2):
        inv = plsc.load_gather(
            den_inv, [jnp.zeros((16,), jnp.int32) + (q * OC + i)])
        for j in range(H // 16):
          v = zbuf[p * OC + i, pl.ds(j * 16, 16)] * inv
          if apply_relu:
            v = jnp.maximum(v, 0.0)
          zbuf[p * OC + i, pl.ds(j * 16, 16)] = v
        return c2
      lax.fori_loop(0, OC, out_row, 0)
      pltpu.sync_copy(zb[p], out.at[pl.ds(c * s_pad + s * rows_t + q * OC,
                                          OC)])

    o_in_start(0, 0, sem_c)
    def out_chunk(q, carry):
      @pl.when(lax.rem(q, 2) == 0)
      def _():
        o_step(q, 0, sem_c, sem_d)
      @pl.when(lax.rem(q, 2) == 1)
      def _():
        o_step(q, 1, sem_d, sem_c)
      return carry
    lax.fori_loop(0, n_oc, out_chunk, 0)

  return seg


_seg_v2e = _make_seg_kernel(NP, NE, has_gval=False, apply_relu=False,
                            chunk=80, dbl=True)
_seg_e2v = _make_seg_kernel(NE, NP, has_gval=True, apply_relu=True,
                            chunk=80, dbl=True)


def _mm_body(x0, x1, w, b, o):
  acc = jnp.dot(x0[...], w[:H, :], preferred_element_type=jnp.float32)
  acc += jnp.dot(x1[...], w[H:, :], preferred_element_type=jnp.float32)
  o[...] = acc + b[...]


def _matmul_halves(xh, w, b):
  mb = 512
  grid = (NP // mb, 2)
  return pl.pallas_call(
      _mm_body,
      grid=grid,
      in_specs=[
          pl.BlockSpec((mb, H), lambda m, h: (m, 0)),
          pl.BlockSpec((mb, H), lambda m, h: (NP // mb + m, 0)),
          pl.BlockSpec((D, H), lambda m, h: (0, h)),
          pl.BlockSpec((1, H), lambda m, h: (0, h)),
      ],
      out_specs=pl.BlockSpec((mb, H), lambda m, h: (h * (NP // mb) + m, 0)),
      out_shape=jax.ShapeDtypeStruct((2 * NP, H), jnp.float32),
  )(xh, xh, w, b.reshape(1, D))


def _edge_body(y, a0, a1, z, e):
  y0 = y[:NE, :]
  y1 = y[NE:, :]
  alpha = (jnp.sum(y0 * a0[...], axis=1, keepdims=True)
           + jnp.sum(y1 * a1[...], axis=1, keepdims=True))
  sc = jnp.where(alpha > 0.0, alpha, 0.2 * alpha)
  ev = jnp.exp(sc - jnp.max(sc))
  z[:NE, :] = y0 * ev
  z[NE:, :] = y1 * ev
  e[...] = ev


def _edge_call(y, a):
  zh, e = pl.pallas_call(
      _edge_body,
      in_specs=[
          pl.BlockSpec((2 * NE, H), lambda: (0, 0)),
          pl.BlockSpec((1, H), lambda: (0, 0)),
          pl.BlockSpec((1, H), lambda: (0, 0)),
      ],
      out_specs=[
          pl.BlockSpec((2 * NE, H), lambda: (0, 0)),
          pl.BlockSpec((NE, 1), lambda: (0, 0)),
      ],
      out_shape=[
          jax.ShapeDtypeStruct((2 * NE, H), jnp.float32),
          jax.ShapeDtypeStruct((NE, 1), jnp.float32),
      ],
  )(y, a[:H].reshape(1, H), a[H:].reshape(1, H))
  return zh, e.reshape(NE)


def _layer(xh, nig4, ni4, eig4, ei4, gdummy, w, b, a):
  xph = _matmul_halves(xh, w, b)
  yh, _ = _seg_v2e(xph, nig4, ei4, gdummy)
  zh, e = _edge_call(yh, a)
  xo, _ = _seg_e2v(zh, eig4, ni4, e)
  return xo


def kernel(x, node_idx, edge_idx, W1, b1, a1, W2, b2, a2):
  ni4 = node_idx.astype(jnp.int32).reshape(NSEC, TILES, SEC)
  ei4 = edge_idx.astype(jnp.int32).reshape(NSEC, TILES, SEC)
  nig4 = jnp.stack([ni4, ni4 + NP])
  eig4 = jnp.stack([ei4, ei4 + NE])
  pad = jnp.zeros((NP - NN, H), jnp.float32)
  xh = jnp.concatenate([x[:, :H], pad, x[:, H:], pad], axis=0)
  gdummy = jnp.zeros((NE,), jnp.float32)

  h = _layer(xh, nig4, ni4, eig4, ei4, gdummy, W1, b1, a1)
  h = _layer(h, nig4, ni4, eig4, ei4, gdummy, W2, b2, a2)
  return jnp.concatenate([h[:NN, :], h[NP:NP + NN, :]], axis=1)


# speedup vs baseline: 1.2124x; 1.2011x over previous
import functools

import jax
import jax.numpy as jnp
from jax import lax
from jax.experimental import pallas as pl
from jax.experimental.pallas import tpu as pltpu
from jax.experimental.pallas import tpu_sc as plsc

NN = 10000
NE = 2048
NNZ = 160000
D = 256
H = 128
NP = 10240
TILES = 16
PER_TILE = NNZ // TILES
CHUNK = 80
SEC = 2000
NSEC = PER_TILE // SEC
OC = 32


def _make_seg_kernel(n_rows, s_pad, has_gval, apply_relu, chunk, dbl):
  rows_t = s_pad // TILES
  n_oc = rows_t // OC
  sec_chunks = SEC // chunk

  mesh = plsc.VectorSubcoreMesh(core_axis_name="c", subcore_axis_name="s")

  @functools.partial(
      pl.kernel,
      out_type=(jax.ShapeDtypeStruct((2 * s_pad, H), jnp.float32),
                jax.ShapeDtypeStruct((2, TILES, s_pad), jnp.float32)),
      mesh=mesh,
      compiler_params=pltpu.CompilerParams(needs_layout_passes=False),
      scratch_types=[
          pltpu.VMEM((SEC,), jnp.int32),
          pltpu.VMEM((SEC,), jnp.int32),
          pltpu.VMEM(((2 if dbl else 1) * chunk, H), jnp.float32),
          pltpu.VMEM((2 * OC, H), jnp.float32),
          pltpu.VMEM((s_pad,), jnp.float32),
          pltpu.VMEM((rows_t,), jnp.float32),
          pltpu.VMEM((rows_t,), jnp.float32),
      ] + ([pltpu.VMEM((NE,), jnp.float32)] if has_gval else []) + [
          pltpu.VMEM_SHARED((s_pad, H), jnp.float32),
          pltpu.SemaphoreType.DMA,
          pltpu.SemaphoreType.DMA,
          pltpu.SemaphoreType.DMA,
          pltpu.SemaphoreType.DMA,
      ],
  )
  def seg(table, idxg4, idxs4, *args):
    (gval, out, scal_pub,
     idxg_v, idxs_v, rows_v, zbuf, scal_l, den_inv, tmp_v) = args[:10]
    rest = args[10:]
    if has_gval:
      gval_v, acc, sem_a, sem_b, sem_sa, sem_sb = rest
    else:
      acc, sem_a, sem_b, sem_sa, sem_sb = rest
      gval_v = None
    sem_c, sem_d = sem_a, sem_b
    c = lax.axis_index("c")
    s = lax.axis_index("s")
    z16 = jnp.zeros((16,), jnp.float32)

    def zero_zbuf(i, carry):
      for j in range(H // 16):
        zbuf[i, pl.ds(j * 16, 16)] = z16
      return carry
    lax.fori_loop(0, OC, zero_zbuf, 0)

    def zero_scal(i, carry):
      scal_l[pl.ds(i * 16, 16)] = z16
      return carry
    lax.fori_loop(0, s_pad // 16, zero_scal, 0)

    def zero_acc(q, carry):
      pltpu.sync_copy(zbuf.at[pl.ds(0, OC)],
                      acc.at[pl.ds(s * rows_t + q * OC, OC)])
      return carry
    lax.fori_loop(0, n_oc, zero_acc, 0)
    plsc.subcore_barrier()

    if has_gval:
      pltpu.sync_copy(gval, gval_v)
    goff = c * n_rows

    def g_start(k, buf, sem):
      pltpu.async_copy(table.at[idxg_v.at[pl.ds(k * chunk, chunk)]],
                       buf, sem)

    def g_wait(buf, sem):
      pltpu.make_async_copy(table.at[idxg_v.at[pl.ds(0, chunk)]],
                            buf, sem).wait()

    def s_start(k, buf, sem):
      pltpu.async_copy(buf, acc.at[idxs_v.at[pl.ds(k * chunk, chunk)]],
                       sem, add=True)

    def s_wait(buf, sem):
      pltpu.make_async_copy(buf, acc.at[idxs_v.at[pl.ds(0, chunk)]],
                            sem).wait()

    buf_a = rows_v.at[pl.ds(0, chunk)]
    buf_b = rows_v.at[pl.ds(chunk, chunk)] if dbl else None

    def section(sec):
      pltpu.sync_copy(idxg4.at[c, sec, s], idxg_v)
      pltpu.sync_copy(idxs4.at[sec, s], idxs_v)
      if dbl:
        g_start(0, buf_a, sem_a)

      def row_chunk(k, carry2):
        for j in range(chunk // 16):
          si = idxs_v[pl.ds(k * chunk + j * 16, 16)]
          if has_gval:
            gi = idxg_v[pl.ds(k * chunk + j * 16, 16)] - goff
            vals = plsc.load_gather(gval_v, [gi])
          else:
            vals = jnp.ones((16,), jnp.float32)
          plsc.addupdate_scatter(scal_l, [si], vals)

        if dbl:
          def step(buf, gsem, ssem, obuf, ogsem, ossem):
            @pl.when(k >= 1)
            def _():
              s_wait(obuf, ossem)
            @pl.when(k + 1 < sec_chunks)
            def _():
              g_start(k + 1, obuf, ogsem)
            g_wait(buf, gsem)
            s_start(k, buf, ssem)
          @pl.when(lax.rem(k, 2) == 0)
          def _():
            step(buf_a, sem_a, sem_sa, buf_b, sem_b, sem_sb)
          @pl.when(lax.rem(k, 2) == 1)
          def _():
            step(buf_b, sem_b, sem_sb, buf_a, sem_a, sem_sa)
        else:
          g_start(k, rows_v, sem_a)
          g_wait(rows_v, sem_a)
          s_start(k, rows_v, sem_sa)
          s_wait(rows_v, sem_sa)
        return carry2
      lax.fori_loop(0, sec_chunks, row_chunk, 0)
      if dbl:
        if (sec_chunks - 1) % 2 == 0:
          s_wait(buf_a, sem_sa)
        else:
          s_wait(buf_b, sem_sb)

    for sec in range(NSEC):
      section(sec)

    pltpu.sync_copy(scal_l, scal_pub.at[c, s])

    plsc.subcore_barrier()

    pltpu.sync_copy(scal_pub.at[c, 0, pl.ds(s * rows_t, rows_t)], den_inv)
    for t2 in range(1, TILES):
      pltpu.sync_copy(scal_pub.at[c, t2, pl.ds(s * rows_t, rows_t)], tmp_v)
      def red(i, carry):
        den_inv[pl.ds(i * 16, 16)] = (den_inv[pl.ds(i * 16, 16)]
                                      + tmp_v[pl.ds(i * 16, 16)])
        return carry
      lax.fori_loop(0, rows_t // 16, red, 0)
    def inv_loop(i, carry):
      dv = den_inv[pl.ds(i * 16, 16)]
      den_inv[pl.ds(i * 16, 16)] = 1.0 / jnp.where(dv > 0.0, dv, 1.0)
      return carry
    lax.fori_loop(0, rows_t // 16, inv_loop, 0)

    zb = [zbuf.at[pl.ds(0, OC)], zbuf.at[pl.ds(OC, OC)]]

    def o_in_start(q, p, sem):
      pltpu.async_copy(acc.at[pl.ds(s * rows_t + q * OC, OC)], zb[p], sem)

    def o_in_wait(p, sem):
      pltpu.make_async_copy(acc.at[pl.ds(0, OC)], zb[p], sem).wait()

    def o_step(q, p, sem, osem):
      o_in_wait(p, sem)
      @pl.when(q + 1 < n_oc)
      def _():
        o_in_start(q + 1, 1 - p, osem)
      def out_row(i, c2):
        inv = plsc.load_gather(
            den_inv, [jnp.zeros((16,), jnp.int32) + (q * OC + i)])
        for j in range(H // 16):
          v = zbuf[p * OC + i, pl.ds(j * 16, 16)] * inv
          if apply_relu:
            v = jnp.maximum(v, 0.0)
          zbuf[p * OC + i, pl.ds(j * 16, 16)] = v
        return c2
      lax.fori_loop(0, OC, out_row, 0)
      pltpu.sync_copy(zb[p], out.at[pl.ds(c * s_pad + s * rows_t + q * OC,
                                          OC)])

    o_in_start(0, 0, sem_c)
    def out_chunk(q, carry):
      @pl.when(lax.rem(q, 2) == 0)
      def _():
        o_step(q, 0, sem_c, sem_d)
      @pl.when(lax.rem(q, 2) == 1)
      def _():
        o_step(q, 1, sem_d, sem_c)
      return carry
    lax.fori_loop(0, n_oc, out_chunk, 0)

  return seg


_seg_v2e = _make_seg_kernel(NP, NE, has_gval=False, apply_relu=False,
                            chunk=80, dbl=True)
_seg_e2v = _make_seg_kernel(NE, NP, has_gval=True, apply_relu=True,
                            chunk=80, dbl=True)


def _mm_body(x0, x1, w, b, o):
  acc = jnp.dot(x0[...], w[:H, :], preferred_element_type=jnp.float32)
  acc += jnp.dot(x1[...], w[H:, :], preferred_element_type=jnp.float32)
  o[...] = acc + b[...]


def _matmul_halves(xh, w, b):
  mb = 512
  grid = (NP // mb, 2)
  return pl.pallas_call(
      _mm_body,
      grid=grid,
      in_specs=[
          pl.BlockSpec((mb, H), lambda m, h: (m, 0)),
          pl.BlockSpec((mb, H), lambda m, h: (NP // mb + m, 0)),
          pl.BlockSpec((D, H), lambda m, h: (0, h)),
          pl.BlockSpec((1, H), lambda m, h: (0, h)),
      ],
      out_specs=pl.BlockSpec((mb, H), lambda m, h: (h * (NP // mb) + m, 0)),
      out_shape=jax.ShapeDtypeStruct((2 * NP, H), jnp.float32),
  )(xh, xh, w, b.reshape(1, D))


def _edge_body(y, a0, a1, z, e):
  y0 = y[:NE, :]
  y1 = y[NE:, :]
  alpha = (jnp.sum(y0 * a0[...], axis=1, keepdims=True)
           + jnp.sum(y1 * a1[...], axis=1, keepdims=True))
  sc = jnp.where(alpha > 0.0, alpha, 0.2 * alpha)
  ev = jnp.exp(sc - jnp.max(sc))
  z[:NE, :] = y0 * ev
  z[NE:, :] = y1 * ev
  e[...] = ev


def _edge_call(y, a):
  zh, e = pl.pallas_call(
      _edge_body,
      in_specs=[
          pl.BlockSpec((2 * NE, H), lambda: (0, 0)),
          pl.BlockSpec((1, H), lambda: (0, 0)),
          pl.BlockSpec((1, H), lambda: (0, 0)),
      ],
      out_specs=[
          pl.BlockSpec((2 * NE, H), lambda: (0, 0)),
          pl.BlockSpec((NE, 1), lambda: (0, 0)),
      ],
      out_shape=[
          jax.ShapeDtypeStruct((2 * NE, H), jnp.float32),
          jax.ShapeDtypeStruct((NE, 1), jnp.float32),
      ],
  )(y, a[:H].reshape(1, H), a[H:].reshape(1, H))
  return zh, e.reshape(NE)


def _layer(xh, nig4, ni4, eig4, ei4, gdummy, w, b, a):
  xph = _matmul_halves(xh, w, b)
  yh, _ = _seg_v2e(xph, nig4, ei4, gdummy)
  zh, e = _edge_call(yh, a)
  xo, _ = _seg_e2v(zh, eig4, ni4, e)
  return xo


def kernel(x, node_idx, edge_idx, W1, b1, a1, W2, b2, a2):
  ni4 = node_idx.astype(jnp.int32).reshape(NSEC, TILES, SEC)
  ei4 = edge_idx.astype(jnp.int32).reshape(NSEC, TILES, SEC)
  nig4 = jnp.stack([ni4, ni4 + NP])
  eig4 = jnp.stack([ei4, ei4 + NE])
  pad = jnp.zeros((NP - NN, H), jnp.float32)
  xh = jnp.concatenate([x[:, :H], pad, x[:, H:], pad], axis=0)
  gdummy = jnp.zeros((NE,), jnp.float32)

  h = _layer(xh, nig4, ni4, eig4, ei4, gdummy, W1, b1, a1)
  h = _layer(h, nig4, ni4, eig4, ei4, gdummy, W2, b2, a2)
  return jnp.concatenate([h[:NN, :], h[NP:NP + NN, :]], axis=1)
